# baseline (device time: 24640 ns/iter reference)
import jax
import jax.numpy as jnp
from jax import lax
from jax.experimental import pallas as pl
from jax.experimental.pallas import tpu as pltpu

N_DEV = 4
B, SQ, SKV, HQ, DH = 2, 128, 512, 4, 64
SKV_LOC = SKV // N_DEV
HD = HQ * DH
WINDOW = 128
SCALE = 0.125
NEG = -1e9


def kernel(x, Wq, K_ext, V_ext, Wo):
    K2 = K_ext.reshape(B, SKV_LOC, HD)
    V2 = V_ext.reshape(B, SKV_LOC, HD)

    def body(x_ref, wq_ref, k_ref, v_ref, wo_ref, out_ref,
             kcomm_ref, vcomm_ref, kfull_ref, vfull_ref, ctx_ref,
             ksend, krecv, vsend, vrecv):
        my = lax.axis_index("i")
        left = lax.rem(my + N_DEV - 1, N_DEV)
        right = lax.rem(my + 1, N_DEV)

        barrier_sem = pltpu.get_barrier_semaphore()
        for nbr in (left, right):
            pl.semaphore_signal(
                barrier_sem, inc=1,
                device_id=(nbr,), device_id_type=pl.DeviceIdType.MESH,
            )
        pl.semaphore_wait(barrier_sem, 2)

        kc = k_ref[...].astype(jnp.bfloat16)
        vc = v_ref[...].astype(jnp.bfloat16)
        kcomm_ref[0] = kc
        vcomm_ref[0] = vc
        kfull_ref[:, pl.ds(my * SKV_LOC, SKV_LOC), :] = kc
        vfull_ref[:, pl.ds(my * SKV_LOC, SKV_LOC), :] = vc

        for h in range(N_DEV - 1):
            s, r = h % 2, (h + 1) % 2
            krdma = pltpu.make_async_remote_copy(
                src_ref=kcomm_ref.at[s], dst_ref=kcomm_ref.at[r],
                send_sem=ksend.at[s], recv_sem=krecv.at[r],
                device_id=(right,), device_id_type=pl.DeviceIdType.MESH,
            )
            vrdma = pltpu.make_async_remote_copy(
                src_ref=vcomm_ref.at[s], dst_ref=vcomm_ref.at[r],
                send_sem=vsend.at[s], recv_sem=vrecv.at[r],
                device_id=(right,), device_id_type=pl.DeviceIdType.MESH,
            )
            krdma.start()
            vrdma.start()
            krdma.wait()
            vrdma.wait()
            origin = lax.rem(my - h - 1 + N_DEV, N_DEV)
            kfull_ref[:, pl.ds(origin * SKV_LOC, SKV_LOC), :] = kcomm_ref[r]
            vfull_ref[:, pl.ds(origin * SKV_LOC, SKV_LOC), :] = vcomm_ref[r]

        wq = wq_ref[...].astype(jnp.bfloat16)
        wo = wo_ref[...].astype(jnp.bfloat16)
        qi = lax.broadcasted_iota(jnp.int32, (SQ, SKV), 0)
        ki = lax.broadcasted_iota(jnp.int32, (SQ, SKV), 1)
        mask = jnp.abs(qi - ki) <= WINDOW

        for b in range(B):
            xb = x_ref[b].astype(jnp.bfloat16)
            qb = lax.dot_general(
                xb, wq, (((1,), (0,)), ((), ())),
                preferred_element_type=jnp.float32,
            ).astype(jnp.bfloat16)
            kb = kfull_ref[b]
            vb = vfull_ref[b]
            for h in range(HQ):
                qbh = lax.slice(qb, (0, h * DH), (SQ, (h + 1) * DH))
                kbh = lax.slice(kb, (0, h * DH), (SKV, (h + 1) * DH))
                vbh = lax.slice(vb, (0, h * DH), (SKV, (h + 1) * DH))
                scores = lax.dot_general(
                    qbh, kbh, (((1,), (1,)), ((), ())),
                    preferred_element_type=jnp.float32,
                ) * SCALE
                scores = jnp.where(mask, scores, NEG)
                m = jnp.max(scores, axis=1, keepdims=True)
                w = jnp.exp(scores - m)
                w = w / jnp.sum(w, axis=1, keepdims=True)
                ctx = lax.dot_general(
                    w.astype(jnp.bfloat16), vbh, (((1,), (0,)), ((), ())),
                    preferred_element_type=jnp.float32,
                )
                ctx_ref[b, :, pl.ds(h * DH, DH)] = ctx.astype(jnp.bfloat16)

        for b in range(B):
            out_ref[b] = lax.dot_general(
                ctx_ref[b], wo, (((1,), (0,)), ((), ())),
                preferred_element_type=jnp.float32,
            )

    return pl.pallas_call(
        body,
        out_shape=jax.ShapeDtypeStruct((B, SQ, SKV), jnp.float32),
        in_specs=[pl.BlockSpec(memory_space=pltpu.VMEM)] * 5,
        out_specs=pl.BlockSpec(memory_space=pltpu.VMEM),
        scratch_shapes=[
            pltpu.VMEM((2, B, SKV_LOC, HD), jnp.bfloat16),
            pltpu.VMEM((2, B, SKV_LOC, HD), jnp.bfloat16),
            pltpu.VMEM((B, SKV, HD), jnp.bfloat16),
            pltpu.VMEM((B, SKV, HD), jnp.bfloat16),
            pltpu.VMEM((B, SQ, HD), jnp.bfloat16),
            pltpu.SemaphoreType.DMA((2,)),
            pltpu.SemaphoreType.DMA((2,)),
            pltpu.SemaphoreType.DMA((2,)),
            pltpu.SemaphoreType.DMA((2,)),
        ],
        compiler_params=pltpu.CompilerParams(collective_id=0),
    )(x, Wq, K2, V2, Wo)


# device time: 14928 ns/iter; 1.6506x vs baseline; 1.6506x over previous
import jax
import jax.numpy as jnp
from jax import lax
from jax.experimental import pallas as pl
from jax.experimental.pallas import tpu as pltpu

N_DEV = 4
B, SQ, SKV, HQ, DH = 2, 128, 512, 4, 64
SKV_LOC = SKV // N_DEV
NKV = 2 * SKV_LOC
HD = HQ * DH
WINDOW = 128
SCALE = 0.125

NN = (((1,), (0,)), ((), ()))
NT = (((1,), (1,)), ((), ()))


def kernel(x, Wq, K_ext, V_ext, Wo):
    K2 = K_ext.reshape(B, SKV_LOC, HD)
    V2 = V_ext.reshape(B, SKV_LOC, HD)

    def body(x_ref, wq_ref, k_ref, v_ref, wo_ref, out_ref,
             kv01_ref, ctx_ref, send_sems, recv_sems):
        my = lax.axis_index("i")

        barrier_sem = pltpu.get_barrier_semaphore()

        @pl.when(my == 0)
        def _():
            pl.semaphore_signal(
                barrier_sem, inc=1,
                device_id=(1,), device_id_type=pl.DeviceIdType.MESH)

        @pl.when(my == 1)
        def _():
            pl.semaphore_signal(
                barrier_sem, inc=1,
                device_id=(0,), device_id_type=pl.DeviceIdType.MESH)

        @pl.when(my >= 2)
        def _():
            for tgt in (0, 1):
                pl.semaphore_signal(
                    barrier_sem, inc=1,
                    device_id=(tgt,), device_id_type=pl.DeviceIdType.MESH)

        @pl.when(my < 2)
        def _():
            pl.semaphore_wait(barrier_sem, 3)

        def do_broadcast(slot, targets):
            kv01_ref[slot, 0] = k_ref[...].astype(jnp.bfloat16)
            kv01_ref[slot, 1] = v_ref[...].astype(jnp.bfloat16)
            for j, tgt in enumerate(targets):
                pltpu.make_async_remote_copy(
                    src_ref=kv01_ref.at[slot], dst_ref=kv01_ref.at[slot],
                    send_sem=send_sems.at[j], recv_sem=recv_sems.at[slot],
                    device_id=(tgt,), device_id_type=pl.DeviceIdType.MESH,
                ).start()

        @pl.when(my == 0)
        def _():
            do_broadcast(0, (1, 2, 3))

        @pl.when(my == 1)
        def _():
            do_broadcast(1, (0, 2, 3))

        wq = wq_ref[...].astype(jnp.bfloat16)
        wo = wo_ref[...].astype(jnp.bfloat16)
        qi = lax.broadcasted_iota(jnp.int32, (SQ, NKV), 0)
        ki = lax.broadcasted_iota(jnp.int32, (SQ, NKV), 1)
        mask = (ki - qi) <= WINDOW
        qbs = []
        for b in range(B):
            xb = x_ref[b].astype(jnp.bfloat16)
            qb = lax.dot_general(
                xb, wq, NN, preferred_element_type=jnp.float32) * SCALE
            qbs.append(qb.astype(jnp.bfloat16))

        def wait_recv_slot(s):
            pltpu.make_async_remote_copy(
                src_ref=kv01_ref.at[s], dst_ref=kv01_ref.at[s],
                send_sem=send_sems.at[0], recv_sem=recv_sems.at[s],
                device_id=(0,), device_id_type=pl.DeviceIdType.MESH,
            ).wait_recv()

        @pl.when(my == 0)
        def _():
            wait_recv_slot(1)

        @pl.when(my == 1)
        def _():
            wait_recv_slot(0)

        @pl.when(my >= 2)
        def _():
            wait_recv_slot(0)
            wait_recv_slot(1)

        for b in range(B):
            kb = jnp.concatenate(
                [kv01_ref[0, 0, b], kv01_ref[1, 0, b]], axis=0)
            vb = jnp.concatenate(
                [kv01_ref[0, 1, b], kv01_ref[1, 1, b]], axis=0)
            for h in range(HQ):
                qbh = lax.slice(qbs[b], (0, h * DH), (SQ, (h + 1) * DH))
                kbh = lax.slice(kb, (0, h * DH), (NKV, (h + 1) * DH))
                vbh = lax.slice(vb, (0, h * DH), (NKV, (h + 1) * DH))
                s = lax.dot_general(
                    qbh, kbh, NT, preferred_element_type=jnp.float32)
                w = jnp.where(mask, jnp.exp(s), 0.0)
                w = w / jnp.sum(w, axis=1, keepdims=True)
                ctx = lax.dot_general(
                    w.astype(jnp.bfloat16), vbh, NN,
                    preferred_element_type=jnp.float32)
                ctx_ref[b, :, pl.ds(h * DH, DH)] = ctx.astype(jnp.bfloat16)

        for b in range(B):
            out_ref[b] = lax.dot_general(
                ctx_ref[b], wo, NN, preferred_element_type=jnp.float32)

        def wait_sends(slot):
            for j in range(3):
                pltpu.make_async_remote_copy(
                    src_ref=kv01_ref.at[slot], dst_ref=kv01_ref.at[slot],
                    send_sem=send_sems.at[j], recv_sem=recv_sems.at[slot],
                    device_id=(0,), device_id_type=pl.DeviceIdType.MESH,
                ).wait_send()

        @pl.when(my == 0)
        def _():
            wait_sends(0)

        @pl.when(my == 1)
        def _():
            wait_sends(1)

    return pl.pallas_call(
        body,
        out_shape=jax.ShapeDtypeStruct((B, SQ, SKV), jnp.float32),
        in_specs=[pl.BlockSpec(memory_space=pltpu.VMEM)] * 5,
        out_specs=pl.BlockSpec(memory_space=pltpu.VMEM),
        scratch_shapes=[
            pltpu.VMEM((2, 2, B, SKV_LOC, HD), jnp.bfloat16),
            pltpu.VMEM((B, SQ, HD), jnp.bfloat16),
            pltpu.SemaphoreType.DMA((3,)),
            pltpu.SemaphoreType.DMA((2,)),
        ],
        compiler_params=pltpu.CompilerParams(collective_id=0),
    )(x, Wq, K2, V2, Wo)
